# baseline (speedup 1.0000x reference)
import functools

import jax
import jax.numpy as jnp
from jax import lax
from jax.experimental import pallas as pl
from jax.experimental.pallas import tpu as pltpu
from jax.experimental.pallas import tpu_sc as plsc

B = 16384
D = 64
NC = 2
NS = 16
NW = NC * NS
B_PER_W = B // NW
HALF = B_PER_W // 2
L = 16


def _body(u_hbm, i_hbm, ue_hbm, ie_hbm, out_hbm,
          u_idx, i_idx, gu, gi, ue_rows, ie_rows, out_v, sem_u, sem_i):
    wid = lax.axis_index("s") * NC + lax.axis_index("c")
    base = wid * B_PER_W

    pltpu.sync_copy(u_hbm.at[pl.ds(base, B_PER_W)], u_idx)
    pltpu.sync_copy(i_hbm.at[pl.ds(base, B_PER_W)], i_idx)

    lane = lax.iota(jnp.int32, L)

    def half(h):
        hb = h * HALF
        def mkidx(c, _):
            gu[pl.ds(c * L, L)] = lax.shift_right_logical(
                u_idx[pl.ds(hb + c * L, L)], 1)
            gi[pl.ds(c * L, L)] = lax.shift_right_logical(
                i_idx[pl.ds(hb + c * L, L)], 1)
            return _
        lax.fori_loop(0, HALF // L, mkidx, 0)

        cp_u = pltpu.make_async_copy(ue_hbm.at[gu], ue_rows, sem_u)
        cp_i = pltpu.make_async_copy(ie_hbm.at[gi], ie_rows, sem_i)
        cp_u.start()
        cp_i.start()
        cp_u.wait()
        cp_i.wait()

        def group(g, carry):
            acc = jnp.zeros((L,), jnp.float32)
            uoff16 = (u_idx[pl.ds(hb + g * L, L)] & 1) * D
            ioff16 = (i_idx[pl.ds(hb + g * L, L)] & 1) * D
            for bb in range(L):
                b = g * L + bb
                uoff = uoff16[bb]
                ioff = ioff16[bb]
                prod = (ue_rows[b, pl.ds(uoff, L)]
                        * ie_rows[b, pl.ds(ioff, L)])
                for c in range(1, D // L):
                    prod = prod + (ue_rows[b, pl.ds(uoff + c * L, L)]
                                   * ie_rows[b, pl.ds(ioff + c * L, L)])
                acc = jnp.where(lane == bb, jnp.sum(prod), acc)
            out_v[pl.ds(hb + g * L, L)] = acc
            return carry

        lax.fori_loop(0, HALF // L, group, 0)

    half(0)
    half(1)

    pltpu.sync_copy(out_v, out_hbm.at[pl.ds(base, B_PER_W)])


@jax.jit
def _score(u, i, ue2, ie2):
    mesh = plsc.VectorSubcoreMesh(core_axis_name="c", subcore_axis_name="s")
    f = functools.partial(
        pl.kernel,
        out_type=jax.ShapeDtypeStruct((B,), jnp.float32),
        mesh=mesh,
        compiler_params=pltpu.CompilerParams(needs_layout_passes=False),
        scratch_types=[
            pltpu.VMEM((B_PER_W,), jnp.int32),
            pltpu.VMEM((B_PER_W,), jnp.int32),
            pltpu.VMEM((HALF,), jnp.int32),
            pltpu.VMEM((HALF,), jnp.int32),
            pltpu.VMEM((HALF, 2 * D), jnp.float32),
            pltpu.VMEM((HALF, 2 * D), jnp.float32),
            pltpu.VMEM((B_PER_W,), jnp.float32),
            pltpu.SemaphoreType.DMA,
            pltpu.SemaphoreType.DMA,
        ],
    )(_body)
    return f(u, i, ue2, ie2)


def kernel(u, i, user_emb, item_emb):
    ue2 = user_emb.reshape(-1, 2 * D)
    ie2 = item_emb.reshape(-1, 2 * D)
    return _score(u, i, ue2, ie2)


# speedup vs baseline: 2.2827x; 2.2827x over previous
import functools

import jax
import jax.numpy as jnp
from jax import lax
from jax.experimental import pallas as pl
from jax.experimental.pallas import tpu as pltpu
from jax.experimental.pallas import tpu_sc as plsc

B = 16384
D = 64
NC = 2
NS = 16
NW = NC * NS
B_PER_W = B // NW
L = 16
N_GROUPS = B_PER_W // L


def _start_pair(uet_hbm, iet_hbm, ue_bufs, ie_bufs, sems, ucol, icol, p):
    cu = pltpu.make_async_copy(uet_hbm.at[:, pl.ds(ucol, 128)],
                               ue_bufs[p], sems[2 * p])
    ci = pltpu.make_async_copy(iet_hbm.at[:, pl.ds(icol, 128)],
                               ie_bufs[p], sems[2 * p + 1])
    cu.start()
    ci.start()


def _wait_pair(uet_hbm, iet_hbm, ue_bufs, ie_bufs, sems, p):
    pltpu.make_async_copy(uet_hbm.at[:, pl.ds(0, 128)],
                          ue_bufs[p], sems[2 * p]).wait()
    pltpu.make_async_copy(iet_hbm.at[:, pl.ds(0, 128)],
                          ie_bufs[p], sems[2 * p + 1]).wait()


def _body(u_hbm, i_hbm, uet_hbm, iet_hbm, out_hbm,
          u_idx, i_idx, ue_t0, ue_t1, ie_t0, ie_t1, out_v,
          s0, s1, s2, s3):
    wid = lax.axis_index("s") * NC + lax.axis_index("c")
    base = wid * B_PER_W

    pltpu.sync_copy(u_hbm.at[pl.ds(base, B_PER_W)],
                    u_idx.at[pl.ds(0, B_PER_W)])
    pltpu.sync_copy(i_hbm.at[pl.ds(base, B_PER_W)],
                    i_idx.at[pl.ds(0, B_PER_W)])

    ue_bufs = (ue_t0, ue_t1)
    ie_bufs = (ie_t0, ie_t1)
    sems = (s0, s1, s2, s3)
    lane = lax.iota(jnp.int32, L)

    def col_of(vec16, bb):
        return pl.multiple_of((vec16[bb] >> 7) * 128, 128)

    u16_0 = u_idx[pl.ds(0, L)]
    i16_0 = i_idx[pl.ds(0, L)]
    _start_pair(uet_hbm, iet_hbm, ue_bufs, ie_bufs, sems,
                col_of(u16_0, 0), col_of(i16_0, 0), 0)
    _start_pair(uet_hbm, iet_hbm, ue_bufs, ie_bufs, sems,
                col_of(u16_0, 1), col_of(i16_0, 1), 1)

    def group(g, carry):
        gbase = g * L
        u16 = u_idx[pl.ds(gbase, L)]
        i16 = i_idx[pl.ds(gbase, L)]
        ui16 = u16 & 127
        ii16 = i16 & 127
        acc = jnp.zeros((L,), jnp.float32)
        for bb in range(L):
            p = bb & 1
            _wait_pair(uet_hbm, iet_hbm, ue_bufs, ie_bufs, sems, p)

            ui = ui16[bb]
            ii = ii16[bb]
            uiv = jnp.full((L,), ui, jnp.int32)
            iiv = jnp.full((L,), ii, jnp.int32)
            ueb = ue_bufs[p]
            ieb = ie_bufs[p]
            prod = jnp.zeros((L,), jnp.float32)
            for c in range(D // L):
                rows = c * L + lane
                uv = plsc.load_gather(ueb, [rows, uiv])
                iv = plsc.load_gather(ieb, [rows, iiv])
                prod = prod + uv * iv
            acc = jnp.where(lane == bb, jnp.sum(prod), acc)

            nxt = gbase + bb + 2

            @pl.when(nxt < B_PER_W)
            def _():
                un = u_idx[pl.ds(nxt, L)]
                inx = i_idx[pl.ds(nxt, L)]
                _start_pair(uet_hbm, iet_hbm, ue_bufs, ie_bufs, sems,
                            col_of(un, 0), col_of(inx, 0), p)

        out_v[pl.ds(gbase, L)] = acc
        return carry

    lax.fori_loop(0, N_GROUPS, group, 0)

    pltpu.sync_copy(out_v, out_hbm.at[pl.ds(base, B_PER_W)])


@jax.jit
def _score(u, i, uet, iet):
    mesh = plsc.VectorSubcoreMesh(core_axis_name="c", subcore_axis_name="s")
    f = functools.partial(
        pl.kernel,
        out_type=jax.ShapeDtypeStruct((B,), jnp.float32),
        mesh=mesh,
        compiler_params=pltpu.CompilerParams(needs_layout_passes=False),
        scratch_types=[
            pltpu.VMEM((B_PER_W + L,), jnp.int32),
            pltpu.VMEM((B_PER_W + L,), jnp.int32),
            pltpu.VMEM((D, 128), jnp.float32),
            pltpu.VMEM((D, 128), jnp.float32),
            pltpu.VMEM((D, 128), jnp.float32),
            pltpu.VMEM((D, 128), jnp.float32),
            pltpu.VMEM((B_PER_W,), jnp.float32),
            pltpu.SemaphoreType.DMA,
            pltpu.SemaphoreType.DMA,
            pltpu.SemaphoreType.DMA,
            pltpu.SemaphoreType.DMA,
        ],
    )(_body)
    return f(u, i, uet, iet)


def kernel(u, i, user_emb, item_emb):
    return _score(u, i, user_emb.T, item_emb.T)


# speedup vs baseline: 2.8026x; 1.2277x over previous
import functools

import jax
import jax.numpy as jnp
from jax import lax
from jax.experimental import pallas as pl
from jax.experimental.pallas import tpu as pltpu
from jax.experimental.pallas import tpu_sc as plsc

B = 16384
D = 64
NC = 2
NS = 16
NW = NC * NS
B_PER_W = B // NW
L = 16
N_GROUPS = B_PER_W // L


def _start_pair(uet_hbm, iet_hbm, ue_bufs, ie_bufs, sems, ucol, icol, p):
    cu = pltpu.make_async_copy(uet_hbm.at[:, pl.ds(ucol, 128)],
                               ue_bufs[p], sems[2 * p])
    ci = pltpu.make_async_copy(iet_hbm.at[:, pl.ds(icol, 128)],
                               ie_bufs[p], sems[2 * p + 1])
    cu.start()
    ci.start()


def _wait_pair(uet_hbm, iet_hbm, ue_bufs, ie_bufs, sems, p):
    pltpu.make_async_copy(uet_hbm.at[:, pl.ds(0, 128)],
                          ue_bufs[p], sems[2 * p]).wait()
    pltpu.make_async_copy(iet_hbm.at[:, pl.ds(0, 128)],
                          ie_bufs[p], sems[2 * p + 1]).wait()


def _body(u_hbm, i_hbm, uet_hbm, iet_hbm, out_hbm,
          u_idx, i_idx, ue_t0, ue_t1, ue_t2, ue_t3,
          ie_t0, ie_t1, ie_t2, ie_t3, out_v,
          s0, s1, s2, s3, s4, s5, s6, s7):
    wid = lax.axis_index("s") * NC + lax.axis_index("c")
    base = wid * B_PER_W

    pltpu.sync_copy(u_hbm.at[pl.ds(base, B_PER_W)],
                    u_idx.at[pl.ds(0, B_PER_W)])
    pltpu.sync_copy(i_hbm.at[pl.ds(base, B_PER_W)],
                    i_idx.at[pl.ds(0, B_PER_W)])

    ue_bufs = (ue_t0, ue_t1, ue_t2, ue_t3)
    ie_bufs = (ie_t0, ie_t1, ie_t2, ie_t3)
    sems = (s0, s1, s2, s3, s4, s5, s6, s7)
    lane = lax.iota(jnp.int32, L)

    def col_of(vec16, bb):
        return pl.multiple_of((vec16[bb] >> 7) * 128, 128)

    u16_0 = u_idx[pl.ds(0, L)]
    i16_0 = i_idx[pl.ds(0, L)]
    for p0 in range(4):
        _start_pair(uet_hbm, iet_hbm, ue_bufs, ie_bufs, sems,
                    col_of(u16_0, p0), col_of(i16_0, p0), p0)

    def group(g, carry):
        gbase = g * L
        u16 = u_idx[pl.ds(gbase, L)]
        i16 = i_idx[pl.ds(gbase, L)]
        ui16 = u16 & 127
        ii16 = i16 & 127
        acc = jnp.zeros((L,), jnp.float32)
        for bb in range(L):
            p = bb & 3
            _wait_pair(uet_hbm, iet_hbm, ue_bufs, ie_bufs, sems, p)

            ui = ui16[bb]
            ii = ii16[bb]
            uiv = jnp.full((L,), ui, jnp.int32)
            iiv = jnp.full((L,), ii, jnp.int32)
            ueb = ue_bufs[p]
            ieb = ie_bufs[p]
            prod = jnp.zeros((L,), jnp.float32)
            for c in range(D // L):
                rows = c * L + lane
                uv = plsc.load_gather(ueb, [rows, uiv])
                iv = plsc.load_gather(ieb, [rows, iiv])
                prod = prod + uv * iv
            acc = jnp.where(lane == bb, jnp.sum(prod), acc)

            nxt = gbase + bb + 4

            @pl.when(nxt < B_PER_W)
            def _():
                un = u_idx[pl.ds(nxt, L)]
                inx = i_idx[pl.ds(nxt, L)]
                _start_pair(uet_hbm, iet_hbm, ue_bufs, ie_bufs, sems,
                            col_of(un, 0), col_of(inx, 0), p)

        out_v[pl.ds(gbase, L)] = acc
        return carry

    lax.fori_loop(0, N_GROUPS, group, 0)

    pltpu.sync_copy(out_v, out_hbm.at[pl.ds(base, B_PER_W)])


@jax.jit
def _score(u, i, uet, iet):
    mesh = plsc.VectorSubcoreMesh(core_axis_name="c", subcore_axis_name="s")
    f = functools.partial(
        pl.kernel,
        out_type=jax.ShapeDtypeStruct((B,), jnp.float32),
        mesh=mesh,
        compiler_params=pltpu.CompilerParams(needs_layout_passes=False),
        scratch_types=[
            pltpu.VMEM((B_PER_W + L,), jnp.int32),
            pltpu.VMEM((B_PER_W + L,), jnp.int32),
        ] + [pltpu.VMEM((D, 128), jnp.float32)] * 8
        + [pltpu.VMEM((B_PER_W,), jnp.float32)]
        + [pltpu.SemaphoreType.DMA] * 8,
    )(_body)
    return f(u, i, uet, iet)


def kernel(u, i, user_emb, item_emb):
    return _score(u, i, user_emb.T, item_emb.T)
